# baseline (speedup 1.0000x reference)
import functools

import jax
import jax.numpy as jnp
from jax import lax
from jax.experimental import pallas as pl
from jax.experimental.pallas import tpu as pltpu
from jax.experimental.pallas import tpu_sc as plsc

_B, _C, _H, _W = 16, 3, 512, 512
_NIMG = _B * _C
_TOTAL = _NIMG * _H * _W
_NW = 32
_BAND = _H // _NW
_K_SC = 18
_N = float(_TOTAL)
_CBLK = 128
_NCB = _W // _CBLK


def _bce_weighted(xv, yv, col_f):
    e = jnp.exp(-jnp.abs(xv))
    z = e / (2.0 + e)
    l1p = 2.0 * z * (1.0 + (z * z) * 0.3542)
    loss = jnp.maximum(xv, 0.0) - xv * yv + l1p
    return col_f * loss


def _make_sc_call():
    mesh = plsc.VectorSubcoreMesh(core_axis_name="c", subcore_axis_name="s")

    @functools.partial(
        pl.kernel,
        mesh=mesh,
        out_type=jax.ShapeDtypeStruct((_NW, 16), jnp.float32),
        scratch_types=[
            pltpu.VMEM((_BAND, _W), jnp.float32),
            pltpu.VMEM((_BAND, _W), jnp.float32),
            pltpu.VMEM((_BAND, _W), jnp.float32),
            pltpu.VMEM((_BAND, _W), jnp.float32),
            pltpu.VMEM((16,), jnp.float32),
            pltpu.SemaphoreType.DMA,
            pltpu.SemaphoreType.DMA,
        ],
    )
    def sc_call(x_hbm, y_hbm, out_hbm, xb0, yb0, xb1, yb1, accbuf, sem0, sem1):
        cid = lax.axis_index("c")
        sid = lax.axis_index("s")
        wid = sid * 2 + cid
        r0 = wid * _BAND
        lane_f = lax.iota(jnp.int32, 16).astype(jnp.float32)

        def _start(img, xb, yb, sem):
            b = img // _C
            ch = img - b * _C
            pltpu.make_async_copy(
                x_hbm.at[b, ch, pl.ds(r0, _BAND), :], xb, sem).start()
            pltpu.make_async_copy(
                y_hbm.at[b, ch, pl.ds(r0, _BAND), :], yb, sem).start()

        def _wait(xb, yb, sem):
            pltpu.make_async_copy(
                x_hbm.at[0, 0, pl.ds(0, _BAND), :], xb, sem).wait()
            pltpu.make_async_copy(
                y_hbm.at[0, 0, pl.ds(0, _BAND), :], yb, sem).wait()

        def _compute(xb, yb, accs):
            def row_body(r, accs_in):
                rc = (r0 + r) * _W

                def col_body(i, a_in):
                    a = list(a_in)
                    for u in range(4):
                        cp = (i + u) * 16
                        xv = xb[r, pl.ds(cp, 16)]
                        yv = yb[r, pl.ds(cp, 16)]
                        col_f = jnp.float32(rc + cp) + lane_f
                        a[u] = a[u] + _bce_weighted(xv, yv, col_f)
                    return tuple(a)

                return plsc.parallel_loop(
                    0, _W // 16, 4, unroll=2, carry=accs_in)(col_body)

            return lax.fori_loop(0, _BAND, row_body, accs)

        _start(0, xb0, yb0, sem0)
        zero = jnp.zeros((16,), jnp.float32)

        def outer(k, accs):
            i0 = 2 * k
            _start(i0 + 1, xb1, yb1, sem1)
            _wait(xb0, yb0, sem0)
            accs = _compute(xb0, yb0, accs)

            @pl.when(i0 + 2 < _K_SC)
            def _():
                _start(i0 + 2, xb0, yb0, sem0)

            _wait(xb1, yb1, sem1)
            return _compute(xb1, yb1, accs)

        accs = lax.fori_loop(0, _K_SC // 2, outer, (zero,) * 4)
        accbuf[...] = (accs[0] + accs[1]) + (accs[2] + accs[3])
        pltpu.sync_copy(accbuf, out_hbm.at[wid])

    return sc_call


_sc_call = _make_sc_call()

_N_TC_IMG = _NIMG - _K_SC


def _tc_body(x_ref, y_ref, o_ref):
    j = pl.program_id(0)
    x = x_ref[...]
    y = y_ref[...]
    col = (lax.broadcasted_iota(jnp.int32, x.shape, 2) * _W
           + lax.broadcasted_iota(jnp.int32, x.shape, 3)).astype(jnp.float32)
    loss = (jnp.maximum(x, 0.0) - x * y
            + jnp.log1p(jnp.exp(-jnp.abs(x))))
    s = jnp.sum(col * loss)

    @pl.when(j == 0)
    def _init():
        o_ref[0, 0] = 0.0

    o_ref[0, 0] += s


def _tc_partial(x, y):
    def img_map(j):
        return (_K_SC // _C + j, 0, 0, 0)

    out = pl.pallas_call(
        _tc_body,
        grid=(_N_TC_IMG // _C,),
        in_specs=[
            pl.BlockSpec((1, _C, _H, _W), img_map),
            pl.BlockSpec((1, _C, _H, _W), img_map),
        ],
        out_specs=pl.BlockSpec(memory_space=pltpu.SMEM),
        out_shape=jax.ShapeDtypeStruct((1, 1), jnp.float32),
    )(x, y)
    return out[0, 0]


def kernel(pred_logits, gts, step):
    del step
    sc_partials = _sc_call(pred_logits, gts)
    tc_sum = _tc_partial(pred_logits, gts)
    return (jnp.sum(sc_partials) + tc_sum) * (1.0 / _N)


# speedup vs baseline: 1.0602x; 1.0602x over previous
import functools

import jax
import jax.numpy as jnp
from jax import lax
from jax.experimental import pallas as pl
from jax.experimental.pallas import tpu as pltpu
from jax.experimental.pallas import tpu_sc as plsc

_B, _C, _H, _W = 16, 3, 512, 512
_NIMG = _B * _C
_TOTAL = _NIMG * _H * _W
_NW = 32
_BAND = _H // _NW
_K_SC = 18
_N = float(_TOTAL)


def _bce_weighted(xv, yv, col_f):
    e = jnp.exp(-jnp.abs(xv))
    z = e / (2.0 + e)
    l1p = 2.0 * z * (1.0 + (z * z) * 0.3542)
    loss = jnp.maximum(xv, 0.0) - xv * yv + l1p
    return col_f * loss


def _make_sc_call():
    mesh = plsc.VectorSubcoreMesh(core_axis_name="c", subcore_axis_name="s")

    @functools.partial(
        pl.kernel,
        mesh=mesh,
        out_type=jax.ShapeDtypeStruct((_NW, 16), jnp.float32),
        scratch_types=[
            pltpu.VMEM((_BAND, _W), jnp.float32),
            pltpu.VMEM((_BAND, _W), jnp.float32),
            pltpu.VMEM((_BAND, _W), jnp.float32),
            pltpu.VMEM((_BAND, _W), jnp.float32),
            pltpu.VMEM((16,), jnp.float32),
            pltpu.SemaphoreType.DMA,
            pltpu.SemaphoreType.DMA,
        ],
    )
    def sc_call(x_hbm, y_hbm, out_hbm, xb0, yb0, xb1, yb1, accbuf, sem0, sem1):
        cid = lax.axis_index("c")
        sid = lax.axis_index("s")
        wid = sid * 2 + cid
        r0 = wid * _BAND
        lane_f = lax.iota(jnp.int32, 16).astype(jnp.float32)

        def _start(img, xb, yb, sem):
            b = img // _C
            ch = img - b * _C
            pltpu.make_async_copy(
                x_hbm.at[b, ch, pl.ds(r0, _BAND), :], xb, sem).start()
            pltpu.make_async_copy(
                y_hbm.at[b, ch, pl.ds(r0, _BAND), :], yb, sem).start()

        def _wait(xb, yb, sem):
            pltpu.make_async_copy(
                x_hbm.at[0, 0, pl.ds(0, _BAND), :], xb, sem).wait()
            pltpu.make_async_copy(
                y_hbm.at[0, 0, pl.ds(0, _BAND), :], yb, sem).wait()

        def _compute(xb, yb, accs):
            def row_body(r, accs_in):
                rc = (r0 + r) * _W

                def col_body(i, a_in):
                    a = list(a_in)
                    for u in range(8):
                        cp = i * 128 + u * 16
                        xv = xb[r, pl.ds(cp, 16)]
                        yv = yb[r, pl.ds(cp, 16)]
                        col_f = jnp.float32(rc + cp) + lane_f
                        a[u % 4] = a[u % 4] + _bce_weighted(xv, yv, col_f)
                    return tuple(a)

                return lax.fori_loop(0, _W // 128, col_body, accs_in)

            return lax.fori_loop(0, _BAND, row_body, accs)

        _start(0, xb0, yb0, sem0)
        zero = jnp.zeros((16,), jnp.float32)

        def outer(k, accs):
            i0 = 2 * k
            _start(i0 + 1, xb1, yb1, sem1)
            _wait(xb0, yb0, sem0)
            accs = _compute(xb0, yb0, accs)

            @pl.when(i0 + 2 < _K_SC)
            def _():
                _start(i0 + 2, xb0, yb0, sem0)

            _wait(xb1, yb1, sem1)
            return _compute(xb1, yb1, accs)

        accs = lax.fori_loop(0, _K_SC // 2, outer, (zero,) * 4)
        accbuf[...] = (accs[0] + accs[1]) + (accs[2] + accs[3])
        pltpu.sync_copy(accbuf, out_hbm.at[wid])

    return sc_call


_sc_call = _make_sc_call()

_N_TC_IMG = _NIMG - _K_SC


def _tc_body(x_ref, y_ref, o_ref):
    j = pl.program_id(0)
    x = x_ref[...]
    y = y_ref[...]
    col = (lax.broadcasted_iota(jnp.int32, x.shape, 2) * _W
           + lax.broadcasted_iota(jnp.int32, x.shape, 3)).astype(jnp.float32)
    loss = (jnp.maximum(x, 0.0) - x * y
            + jnp.log1p(jnp.exp(-jnp.abs(x))))
    s = jnp.sum(col * loss)

    @pl.when(j == 0)
    def _init():
        o_ref[0, 0] = 0.0

    o_ref[0, 0] += s


def _tc_partial(x, y):
    def img_map(j):
        return (_K_SC // _C + j, 0, 0, 0)

    out = pl.pallas_call(
        _tc_body,
        grid=(_N_TC_IMG // _C,),
        in_specs=[
            pl.BlockSpec((1, _C, _H, _W), img_map),
            pl.BlockSpec((1, _C, _H, _W), img_map),
        ],
        out_specs=pl.BlockSpec(memory_space=pltpu.SMEM),
        out_shape=jax.ShapeDtypeStruct((1, 1), jnp.float32),
    )(x, y)
    return out[0, 0]


def kernel(pred_logits, gts, step):
    del step
    sc_partials = _sc_call(pred_logits, gts)
    tc_sum = _tc_partial(pred_logits, gts)
    return (jnp.sum(sc_partials) + tc_sum) * (1.0 / _N)
